# Initial kernel scaffold; baseline (speedup 1.0000x reference)
import jax
import jax.numpy as jnp
from jax.experimental import pallas as pl


def kernel(hidden_states, position_ids, q_w, k_w, v_w, o_w, ln1_w, ln2_w, gate_w, eg_w, eu_w, ed_w, sg_w, su_w, sd_w, sgate_w):
    raise NotImplementedError("write your pallas kernel here")


# speedup vs baseline: 1.1438x; 1.1438x over previous
import functools
import math

import jax
import jax.numpy as jnp
from jax.experimental import pallas as pl

H = 1024
NH = 16
NKV = 4
HD = 64
E = 8
K = 2
I = 1024
EPS = 1e-06
THETA = 1000000.0
S = 2048

TS = 256
LN_THETA = math.log(THETA)


def _rot_mat(n):
    i = jax.lax.broadcasted_iota(jnp.int32, (n, n), 0)
    c = jax.lax.broadcasted_iota(jnp.int32, (n, n), 1)
    cm = jnp.remainder(c, HD)
    neg = jnp.logical_and(i == c + HD // 2, cm < HD // 2)
    pos = jnp.logical_and(i == c - HD // 2, cm >= HD // 2)
    return jnp.where(neg, -1.0, 0.0) + jnp.where(pos, 1.0, 0.0)


def _cos_sin(t, n):
    r = jax.lax.broadcasted_iota(jnp.int32, (TS, n), 0).astype(jnp.float32)
    c = jax.lax.broadcasted_iota(jnp.int32, (TS, n), 1)
    fi = jnp.remainder(c, HD // 2).astype(jnp.float32)
    invf = jnp.exp(fi * (-LN_THETA / (HD // 2)))
    ang = (r + t * TS) * invf
    return jnp.cos(ang), jnp.sin(ang)


def _rmsnorm(x, w):
    v = jnp.mean(x * x, axis=-1, keepdims=True)
    return x * jax.lax.rsqrt(v + EPS) * w


def _pre_kernel(x_ref, qw_ref, kw_ref, vw_ref, ln1_ref, q_ref, k_ref, v_ref):
    t = pl.program_id(0)
    xn = _rmsnorm(x_ref[...], ln1_ref[...])
    q = jax.lax.dot_general(xn, qw_ref[...], (((1,), (1,)), ((), ())),
                            preferred_element_type=jnp.float32)
    k = jax.lax.dot_general(xn, kw_ref[...], (((1,), (1,)), ((), ())),
                            preferred_element_type=jnp.float32)
    v = jax.lax.dot_general(xn, vw_ref[...], (((1,), (1,)), ((), ())),
                            preferred_element_type=jnp.float32)
    cos_q, sin_q = _cos_sin(t, NH * HD)
    rq = jnp.dot(q, _rot_mat(NH * HD), preferred_element_type=jnp.float32)
    q_ref[...] = q * cos_q + rq * sin_q
    cos_k, sin_k = _cos_sin(t, NKV * HD)
    rk = jnp.dot(k, _rot_mat(NKV * HD), preferred_element_type=jnp.float32)
    k_ref[...] = k * cos_k + rk * sin_k
    v_ref[...] = v


def _attn_kernel(q_ref, k_ref, v_ref, o_ref):
    q = q_ref[...]
    k = k_ref[...]
    v = v_ref[...]
    for h in range(NH):
        qh = q[:, h * HD:(h + 1) * HD]
        kv = h // (NH // NKV)
        kh = k[:, kv * HD:(kv + 1) * HD]
        vh = v[:, kv * HD:(kv + 1) * HD]
        s = jax.lax.dot_general(qh, kh, (((1,), (1,)), ((), ())),
                                preferred_element_type=jnp.float32)
        s = s * (1.0 / math.sqrt(HD))
        m = jnp.max(s, axis=-1, keepdims=True)
        p = jnp.exp(s - m)
        p = p / jnp.sum(p, axis=-1, keepdims=True)
        o_ref[:, h * HD:(h + 1) * HD] = jnp.dot(
            p, vh, preferred_element_type=jnp.float32)


def _post_kernel(attn_ref, x_ref, ow_ref, ln2_ref, gw_ref,
                 h2_ref, x2_ref, cw_ref):
    o = jax.lax.dot_general(attn_ref[...], ow_ref[...], (((1,), (1,)), ((), ())),
                            preferred_element_type=jnp.float32)
    h2 = x_ref[...] + o
    h2_ref[...] = h2
    x2 = _rmsnorm(h2, ln2_ref[...])
    x2_ref[...] = x2
    logits = jax.lax.dot_general(x2, gw_ref[...], (((1,), (1,)), ((), ())),
                                 preferred_element_type=jnp.float32)
    lm = jnp.max(logits, axis=-1, keepdims=True)
    el = jnp.exp(logits - lm)
    probs = el / jnp.sum(el, axis=-1, keepdims=True)
    iota = jax.lax.broadcasted_iota(jnp.int32, (TS, E), 1)
    m1 = jnp.max(probs, axis=-1, keepdims=True)
    i1 = jnp.min(jnp.where(probs == m1, iota, E), axis=-1, keepdims=True)
    probs2 = jnp.where(iota == i1, -jnp.inf, probs)
    m2 = jnp.max(probs2, axis=-1, keepdims=True)
    i2 = jnp.min(jnp.where(probs2 == m2, iota, E), axis=-1, keepdims=True)
    cw = jnp.where(iota == i1, m1, 0.0) + jnp.where(iota == i2, m2, 0.0)
    cw_ref[...] = cw


def _moe_kernel(x2_ref, cw_ref, eg_ref, eu_ref, ed_ref, out_ref):
    e = pl.program_id(1)
    x2 = x2_ref[...]
    g = jax.lax.dot_general(x2, eg_ref[0], (((1,), (1,)), ((), ())),
                            preferred_element_type=jnp.float32)
    u = jax.lax.dot_general(x2, eu_ref[0], (((1,), (1,)), ((), ())),
                            preferred_element_type=jnp.float32)
    hdn = g * jax.nn.sigmoid(g) * u
    oute = jax.lax.dot_general(hdn, ed_ref[0], (((1,), (1,)), ((), ())),
                               preferred_element_type=jnp.float32)
    lane = jax.lax.broadcasted_iota(jnp.int32, (TS, E), 1)
    w = jnp.sum(jnp.where(lane == e, cw_ref[...], 0.0), axis=1, keepdims=True)
    contrib = w * oute

    @pl.when(e == 0)
    def _():
        out_ref[...] = contrib

    @pl.when(e > 0)
    def _():
        out_ref[...] += contrib


def _final_kernel(x2_ref, h2_ref, moe_ref, sg_ref, su_ref, sd_ref, sgate_ref,
                  out_ref):
    x2 = x2_ref[...]
    g = jax.lax.dot_general(x2, sg_ref[...], (((1,), (1,)), ((), ())),
                            preferred_element_type=jnp.float32)
    u = jax.lax.dot_general(x2, su_ref[...], (((1,), (1,)), ((), ())),
                            preferred_element_type=jnp.float32)
    shared = jax.lax.dot_general(g * jax.nn.sigmoid(g) * u, sd_ref[...],
                                 (((1,), (1,)), ((), ())),
                                 preferred_element_type=jnp.float32)
    gate = jax.nn.sigmoid(
        jax.lax.dot_general(x2, sgate_ref[...], (((1,), (1,)), ((), ())),
                            preferred_element_type=jnp.float32))
    out_ref[...] = h2_ref[...] + moe_ref[...] + gate * shared


def kernel(hidden_states, position_ids, q_w, k_w, v_w, o_w, ln1_w, ln2_w,
           gate_w, eg_w, eu_w, ed_w, sg_w, su_w, sd_w, sgate_w):
    x = hidden_states.reshape(S, H)
    NT = S // TS

    q, k, v = pl.pallas_call(
        _pre_kernel,
        grid=(NT,),
        in_specs=[
            pl.BlockSpec((TS, H), lambda t: (t, 0)),
            pl.BlockSpec((NH * HD, H), lambda t: (0, 0)),
            pl.BlockSpec((NKV * HD, H), lambda t: (0, 0)),
            pl.BlockSpec((NKV * HD, H), lambda t: (0, 0)),
            pl.BlockSpec((H,), lambda t: (0,)),
        ],
        out_specs=[
            pl.BlockSpec((TS, NH * HD), lambda t: (t, 0)),
            pl.BlockSpec((TS, NKV * HD), lambda t: (t, 0)),
            pl.BlockSpec((TS, NKV * HD), lambda t: (t, 0)),
        ],
        out_shape=[
            jax.ShapeDtypeStruct((S, NH * HD), jnp.float32),
            jax.ShapeDtypeStruct((S, NKV * HD), jnp.float32),
            jax.ShapeDtypeStruct((S, NKV * HD), jnp.float32),
        ],
    )(x, q_w, k_w, v_w, ln1_w)

    attn = pl.pallas_call(
        _attn_kernel,
        grid=(NT,),
        in_specs=[
            pl.BlockSpec((TS, NH * HD), lambda t: (t, 0)),
            pl.BlockSpec((S, NKV * HD), lambda t: (0, 0)),
            pl.BlockSpec((S, NKV * HD), lambda t: (0, 0)),
        ],
        out_specs=pl.BlockSpec((TS, NH * HD), lambda t: (t, 0)),
        out_shape=jax.ShapeDtypeStruct((S, NH * HD), jnp.float32),
    )(q, k, v)

    h2, x2, cw = pl.pallas_call(
        _post_kernel,
        grid=(NT,),
        in_specs=[
            pl.BlockSpec((TS, NH * HD), lambda t: (t, 0)),
            pl.BlockSpec((TS, H), lambda t: (t, 0)),
            pl.BlockSpec((H, NH * HD), lambda t: (0, 0)),
            pl.BlockSpec((H,), lambda t: (0,)),
            pl.BlockSpec((E, H), lambda t: (0, 0)),
        ],
        out_specs=[
            pl.BlockSpec((TS, H), lambda t: (t, 0)),
            pl.BlockSpec((TS, H), lambda t: (t, 0)),
            pl.BlockSpec((TS, E), lambda t: (t, 0)),
        ],
        out_shape=[
            jax.ShapeDtypeStruct((S, H), jnp.float32),
            jax.ShapeDtypeStruct((S, H), jnp.float32),
            jax.ShapeDtypeStruct((S, E), jnp.float32),
        ],
    )(attn, x, o_w, ln2_w, gate_w)

    moe = pl.pallas_call(
        _moe_kernel,
        grid=(NT, E),
        in_specs=[
            pl.BlockSpec((TS, H), lambda t, e: (t, 0)),
            pl.BlockSpec((TS, E), lambda t, e: (t, 0)),
            pl.BlockSpec((1, I, H), lambda t, e: (e, 0, 0)),
            pl.BlockSpec((1, I, H), lambda t, e: (e, 0, 0)),
            pl.BlockSpec((1, H, I), lambda t, e: (e, 0, 0)),
        ],
        out_specs=pl.BlockSpec((TS, H), lambda t, e: (t, 0)),
        out_shape=jax.ShapeDtypeStruct((S, H), jnp.float32),
    )(x2, cw, eg_w, eu_w, ed_w)

    out = pl.pallas_call(
        _final_kernel,
        grid=(NT,),
        in_specs=[
            pl.BlockSpec((TS, H), lambda t: (t, 0)),
            pl.BlockSpec((TS, H), lambda t: (t, 0)),
            pl.BlockSpec((TS, H), lambda t: (t, 0)),
            pl.BlockSpec((I, H), lambda t: (0, 0)),
            pl.BlockSpec((I, H), lambda t: (0, 0)),
            pl.BlockSpec((H, I), lambda t: (0, 0)),
            pl.BlockSpec((1, H), lambda t: (0, 0)),
        ],
        out_specs=pl.BlockSpec((TS, H), lambda t: (t, 0)),
        out_shape=jax.ShapeDtypeStruct((S, H), jnp.float32),
    )(x2, h2, moe, sg_w, su_w, sd_w, sgate_w)

    return out.reshape(1, S, H)
